# Initial kernel scaffold; baseline (speedup 1.0000x reference)
import jax
import jax.numpy as jnp
from jax import lax
from jax.experimental import pallas as pl
from jax.experimental.pallas import tpu as pltpu
from jax.experimental.pallas import tpu_sc as plsc

B, C, N, M = 16, 256, 16384, 4096
NC, NS, L = 2, 16, 16
NW = NC * NS
ROWS_PER_W = (B * C) // NW
C_PER_W = C // (NW // B)


def _gather_body(features_hbm, indices_hbm, out_hbm, idx_v, row_v, res_v, sem):
    wid = lax.axis_index("s") * NC + lax.axis_index("c")
    b = wid // (NW // B)
    c0 = (wid % (NW // B)) * C_PER_W

    pltpu.sync_copy(indices_hbm.at[b], idx_v)

    def row_step(r, carry):
        c = c0 + r
        pltpu.async_copy(features_hbm.at[b, c], row_v, sem).wait()

        def gather_step(j, inner):
            iv = idx_v[pl.ds(j * L, L)]
            res_v[pl.ds(j * L, L)] = plsc.load_gather(row_v, [iv])
            return inner

        lax.fori_loop(0, M // L, gather_step, 0)
        pltpu.sync_copy(res_v, out_hbm.at[b, c])
        return carry

    lax.fori_loop(0, ROWS_PER_W, row_step, 0)


@jax.jit
def kernel(features, indices):
    idx32 = indices.astype(jnp.int32)
    mesh = plsc.VectorSubcoreMesh(core_axis_name="c", subcore_axis_name="s")
    run = pl.kernel(
        _gather_body,
        out_type=jax.ShapeDtypeStruct((B, C, M), jnp.float32),
        mesh=mesh,
        scratch_types=[
            pltpu.VMEM((M,), jnp.int32),
            pltpu.VMEM((N,), jnp.float32),
            pltpu.VMEM((M,), jnp.float32),
            pltpu.SemaphoreType.DMA,
        ],
    )
    return run(features, idx32)


# speedup vs baseline: 2.0162x; 2.0162x over previous
import jax
import jax.numpy as jnp
from jax import lax
from jax.experimental import pallas as pl
from jax.experimental.pallas import tpu as pltpu
from jax.experimental.pallas import tpu_sc as plsc

B, C, N, M = 16, 256, 16384, 4096
NC, NS, L = 2, 16, 16
NW = NC * NS
ROWS_PER_W = (B * C) // NW
C_PER_W = C // (NW // B)


def _gather_body(features_hbm, indices_hbm, out_hbm, idx_v, row_v, res_v, sem):
    wid = lax.axis_index("s") * NC + lax.axis_index("c")
    b = wid // (NW // B)
    c0 = (wid % (NW // B)) * C_PER_W

    pltpu.sync_copy(indices_hbm.at[b], idx_v)

    def row_step(r, carry):
        c = c0 + r
        pltpu.async_copy(features_hbm.at[b, c], row_v, sem).wait()

        def gather_step(j, inner):
            iv = idx_v[pl.ds(j * L, L)]
            res_v[pl.ds(j * L, L)] = plsc.load_gather(row_v, [iv])
            return inner

        lax.fori_loop(0, M // L, gather_step, 0)
        pltpu.sync_copy(res_v, out_hbm.at[b, c])
        return carry

    lax.fori_loop(0, ROWS_PER_W, row_step, 0)


@jax.jit
def kernel(features, indices):
    idx32 = indices.astype(jnp.int32)
    mesh = plsc.VectorSubcoreMesh(core_axis_name="c", subcore_axis_name="s")
    run = pl.kernel(
        _gather_body,
        out_type=jax.ShapeDtypeStruct((B, C, M), jnp.float32),
        mesh=mesh,
        scratch_types=[
            pltpu.VMEM((M,), jnp.int32),
            pltpu.VMEM((N,), jnp.float32),
            pltpu.VMEM((M,), jnp.float32),
            pltpu.SemaphoreType.DMA,
        ],
        compiler_params=pltpu.CompilerParams(needs_layout_passes=False),
    )
    return run(features, idx32)


# speedup vs baseline: 5.3840x; 2.6704x over previous
import jax
import jax.numpy as jnp
from jax import lax
from jax.experimental import pallas as pl
from jax.experimental.pallas import tpu as pltpu
from jax.experimental.pallas import tpu_sc as plsc

B, C, N, M = 16, 256, 16384, 4096
NC, NS, L = 2, 16, 16
NW = NC * NS
ROWS_PER_W = (B * C) // NW
C_PER_W = C // (NW // B)
NBUF = 2


def _gather_body(features_hbm, indices_hbm, out_hbm, idx_v, row_v, res_v,
                 sin0, sin1, sout0, sout1):
    wid = lax.axis_index("s") * NC + lax.axis_index("c")
    b = wid // (NW // B)
    c0 = (wid % (NW // B)) * C_PER_W
    sins = (sin0, sin1)
    souts = (sout0, sout1)

    pltpu.sync_copy(indices_hbm.at[b], idx_v)

    for buf in range(NBUF):
        pltpu.async_copy(features_hbm.at[b, c0 + buf],
                         row_v.at[pl.ds(buf * N, N)], sins[buf])
        pltpu.async_copy(res_v.at[pl.ds(buf * M, M)],
                         out_hbm.at[b, c0 + buf], souts[buf])

    def pair_step(i, carry):
        r0 = i * NBUF
        for buf in range(NBUF):
            r = r0 + buf
            c = c0 + r
            pltpu.make_async_copy(
                features_hbm.at[b, c], row_v.at[pl.ds(buf * N, N)],
                sins[buf]).wait()
            pltpu.make_async_copy(
                res_v.at[pl.ds(buf * M, M)], out_hbm.at[b, c],
                souts[buf]).wait()

            @plsc.parallel_loop(0, M // L, unroll=8)
            def _gather(j):
                iv = idx_v[pl.ds(j * L, L)] + (buf * N)
                res_v[pl.ds(buf * M + j * L, L)] = plsc.load_gather(
                    row_v, [iv])

            pltpu.async_copy(res_v.at[pl.ds(buf * M, M)], out_hbm.at[b, c],
                             souts[buf])
            rn = jnp.minimum(r + NBUF, ROWS_PER_W - 1)
            pltpu.async_copy(features_hbm.at[b, c0 + rn],
                             row_v.at[pl.ds(buf * N, N)], sins[buf])
        return carry

    lax.fori_loop(0, ROWS_PER_W // NBUF, pair_step, 0)

    for buf in range(NBUF):
        pltpu.make_async_copy(
            features_hbm.at[b, c0], row_v.at[pl.ds(buf * N, N)],
            sins[buf]).wait()
        pltpu.make_async_copy(
            res_v.at[pl.ds(buf * M, M)], out_hbm.at[b, c0],
            souts[buf]).wait()


@jax.jit
def kernel(features, indices):
    idx32 = indices.astype(jnp.int32)
    mesh = plsc.VectorSubcoreMesh(core_axis_name="c", subcore_axis_name="s")
    run = pl.kernel(
        _gather_body,
        out_type=jax.ShapeDtypeStruct((B, C, M), jnp.float32),
        mesh=mesh,
        scratch_types=[
            pltpu.VMEM((M,), jnp.int32),
            pltpu.VMEM((NBUF * N,), jnp.float32),
            pltpu.VMEM((NBUF * M,), jnp.float32),
            pltpu.SemaphoreType.DMA,
            pltpu.SemaphoreType.DMA,
            pltpu.SemaphoreType.DMA,
            pltpu.SemaphoreType.DMA,
        ],
        compiler_params=pltpu.CompilerParams(needs_layout_passes=False),
    )
    return run(features, idx32)
